# speedup vs baseline: 1.0058x; 1.0058x over previous
import functools

import jax
import jax.numpy as jnp
from jax.experimental import pallas as pl
from jax.experimental.pallas import tpu as pltpu

GS = 240


def _xsxt_body(x_ref, w_ref, o_ref):
    o_ref[...] = jnp.dot(x_ref[...], w_ref[...],
                         preferred_element_type=jnp.float32)


def _edge_body(s_ref, ea_ref, we_ref, eb1_ref, ew2_ref, eb2_ref, ng_ref,
               nb_ref, o_ref):
    h = s_ref[...] + jnp.dot(ea_ref[...], we_ref[...],
                             preferred_element_type=jnp.float32) + eb1_ref[...]
    h = jnp.maximum(h, 0.0)
    h = jnp.dot(h, ew2_ref[...], preferred_element_type=jnp.float32) + eb2_ref[...]
    h = jnp.maximum(h, 0.0)
    mu = jnp.mean(h, axis=-1, keepdims=True)
    d = h - mu
    var = jnp.mean(d * d, axis=-1, keepdims=True)
    o_ref[...] = d * jax.lax.rsqrt(var + 1e-5) * ng_ref[...] + nb_ref[...]


def _node_body(x_ref, agg_ref, wx_ref, wa_ref, wl_ref, nb1_ref, nw2_ref,
               nb2_ref, o_ref):
    agg = agg_ref[...]
    x = x_ref[...]
    lat = jnp.mean(agg, axis=0, keepdims=True)
    h = (jnp.dot(x, wx_ref[...], preferred_element_type=jnp.float32)
         + jnp.dot(agg, wa_ref[...], preferred_element_type=jnp.float32)
         + jnp.dot(lat, wl_ref[...], preferred_element_type=jnp.float32)
         + nb1_ref[...])
    h = jnp.maximum(h, 0.0)
    o = jnp.dot(h, nw2_ref[...], preferred_element_type=jnp.float32) + nb2_ref[...]
    o_ref[...] = o + x


def kernel(x, edge_index, edge_attr, ew1, eb1, ew2, eb2, ng, nb,
           nw1, nb1, nw2, nb2):
    n, inf = x.shape
    e, ein = edge_attr.shape
    hnf = ew2.shape[1]
    onf = nw2.shape[1]
    row = edge_index[0]
    col = edge_index[1]

    wst = jnp.concatenate([ew1[:inf], ew1[inf:2 * inf]], axis=1)
    we = ew1[2 * inf:]
    wx = nw1[:inf]
    wa = nw1[inf:inf + hnf]
    wl = nw1[inf + hnf:]

    eb1r = eb1.reshape(1, hnf)
    eb2r = eb2.reshape(1, hnf)
    ngr = ng.reshape(1, hnf)
    nbr = nb.reshape(1, hnf)
    nb1r = nb1.reshape(1, hnf)
    nb2r = nb2.reshape(1, onf)

    xsxt = pl.pallas_call(
        _xsxt_body,
        grid=(n // GS,),
        in_specs=[pl.BlockSpec((GS, inf), lambda i: (i, 0)),
                  pl.BlockSpec((inf, 2 * hnf), lambda i: (0, 0))],
        out_specs=pl.BlockSpec((GS, 2 * hnf), lambda i: (i, 0)),
        out_shape=jax.ShapeDtypeStruct((n, 2 * hnf), jnp.float32),
    )(x, wst)

    s = jnp.take(xsxt[:, :hnf], row, axis=0) + jnp.take(xsxt[:, hnf:], col, axis=0)

    be = 2000
    bcast = lambda i: (0, 0)
    edge_feat = pl.pallas_call(
        _edge_body,
        grid=(e // be,),
        in_specs=[pl.BlockSpec((be, hnf), lambda i: (i, 0)),
                  pl.BlockSpec((be, ein), lambda i: (i, 0)),
                  pl.BlockSpec((ein, hnf), bcast),
                  pl.BlockSpec((1, hnf), bcast),
                  pl.BlockSpec((hnf, hnf), bcast),
                  pl.BlockSpec((1, hnf), bcast),
                  pl.BlockSpec((1, hnf), bcast),
                  pl.BlockSpec((1, hnf), bcast)],
        out_specs=pl.BlockSpec((be, hnf), lambda i: (i, 0)),
        out_shape=jax.ShapeDtypeStruct((e, hnf), jnp.float32),
    )(s, edge_attr, we, eb1r, ew2, eb2r, ngr, nbr)

    agg = jax.ops.segment_sum(edge_feat, row, num_segments=n)

    out = pl.pallas_call(
        _node_body,
        grid=(n // GS,),
        in_specs=[pl.BlockSpec((GS, inf), lambda i: (i, 0)),
                  pl.BlockSpec((GS, hnf), lambda i: (i, 0)),
                  pl.BlockSpec((inf, hnf), bcast),
                  pl.BlockSpec((hnf, hnf), bcast),
                  pl.BlockSpec((hnf, hnf), bcast),
                  pl.BlockSpec((1, hnf), bcast),
                  pl.BlockSpec((hnf, onf), bcast),
                  pl.BlockSpec((1, onf), bcast)],
        out_specs=pl.BlockSpec((GS, onf), lambda i: (i, 0)),
        out_shape=jax.ShapeDtypeStruct((n, onf), jnp.float32),
    )(x, agg, wx, wa, wl, nb1r, nw2, nb2r)

    return (out, edge_feat)


# speedup vs baseline: 1.8809x; 1.8701x over previous
import functools

import jax
import jax.numpy as jnp
from jax import lax
from jax.experimental import pallas as pl
from jax.experimental.pallas import tpu as pltpu
from jax.experimental.pallas import tpu_sc as plsc

GS = 240

NC = 2
NSUB = 16
NW = NC * NSUB
CH = 128


def _xsxt_body(x_ref, ws_ref, wt_ref, os_ref, ot_ref):
    os_ref[...] = jnp.dot(x_ref[...], ws_ref[...],
                          preferred_element_type=jnp.float32)
    ot_ref[...] = jnp.dot(x_ref[...], wt_ref[...],
                          preferred_element_type=jnp.float32)


def _make_sc_gather(e_pad, hnf):
    per_w = e_pad // NW
    nch = per_w // CH

    def body(xs_hbm, xt_hbm, row_hbm, col_hbm, out_hbm,
             ridx_v, cidx_v, a_v, b_v, sem_a, sem_b):
        wid = lax.axis_index("s") * NC + lax.axis_index("c")
        base = wid * per_w

        def chunk(ci, carry):
            off = base + ci * CH
            pltpu.sync_copy(row_hbm.at[pl.ds(off, CH)], ridx_v)
            pltpu.sync_copy(col_hbm.at[pl.ds(off, CH)], cidx_v)
            cpa = pltpu.async_copy(xs_hbm.at[ridx_v], a_v, sem_a)
            cpb = pltpu.async_copy(xt_hbm.at[cidx_v], b_v, sem_b)
            cpa.wait()
            cpb.wait()

            def add_row(r, c2):
                for c in range(hnf // 16):
                    sl = pl.ds(c * 16, 16)
                    a_v[r, sl] = a_v[r, sl] + b_v[r, sl]
                return c2

            lax.fori_loop(0, CH, add_row, 0)
            pltpu.sync_copy(a_v, out_hbm.at[pl.ds(off, CH)])
            return carry

        lax.fori_loop(0, nch, chunk, 0)

    mesh = plsc.VectorSubcoreMesh(core_axis_name="c", subcore_axis_name="s",
                                  num_cores=NC, num_subcores=NSUB)
    return pl.kernel(
        body,
        out_type=jax.ShapeDtypeStruct((e_pad, hnf), jnp.float32),
        mesh=mesh,
        scratch_types=[
            pltpu.VMEM((CH,), jnp.int32),
            pltpu.VMEM((CH,), jnp.int32),
            pltpu.VMEM((CH, hnf), jnp.float32),
            pltpu.VMEM((CH, hnf), jnp.float32),
            pltpu.SemaphoreType.DMA,
            pltpu.SemaphoreType.DMA,
        ],
    )


def _edge_body(s_ref, ea_ref, we_ref, eb1_ref, ew2_ref, eb2_ref, ng_ref,
               nb_ref, o_ref):
    h = s_ref[...] + jnp.dot(ea_ref[...], we_ref[...],
                             preferred_element_type=jnp.float32) + eb1_ref[...]
    h = jnp.maximum(h, 0.0)
    h = jnp.dot(h, ew2_ref[...], preferred_element_type=jnp.float32) + eb2_ref[...]
    h = jnp.maximum(h, 0.0)
    mu = jnp.mean(h, axis=-1, keepdims=True)
    d = h - mu
    var = jnp.mean(d * d, axis=-1, keepdims=True)
    o_ref[...] = d * jax.lax.rsqrt(var + 1e-5) * ng_ref[...] + nb_ref[...]


def _node_body(x_ref, agg_ref, wx_ref, wa_ref, wl_ref, nb1_ref, nw2_ref,
               nb2_ref, o_ref):
    agg = agg_ref[...]
    x = x_ref[...]
    lat = jnp.mean(agg, axis=0, keepdims=True)
    h = (jnp.dot(x, wx_ref[...], preferred_element_type=jnp.float32)
         + jnp.dot(agg, wa_ref[...], preferred_element_type=jnp.float32)
         + jnp.dot(lat, wl_ref[...], preferred_element_type=jnp.float32)
         + nb1_ref[...])
    h = jnp.maximum(h, 0.0)
    o = jnp.dot(h, nw2_ref[...], preferred_element_type=jnp.float32) + nb2_ref[...]
    o_ref[...] = o + x


def kernel(x, edge_index, edge_attr, ew1, eb1, ew2, eb2, ng, nb,
           nw1, nb1, nw2, nb2):
    n, inf = x.shape
    e, ein = edge_attr.shape
    hnf = ew2.shape[1]
    onf = nw2.shape[1]
    row = edge_index[0]
    col = edge_index[1]

    we = ew1[2 * inf:]
    wx = nw1[:inf]
    wa = nw1[inf:inf + hnf]
    wl = nw1[inf + hnf:]

    eb1r = eb1.reshape(1, hnf)
    eb2r = eb2.reshape(1, hnf)
    ngr = ng.reshape(1, hnf)
    nbr = nb.reshape(1, hnf)
    nb1r = nb1.reshape(1, hnf)
    nb2r = nb2.reshape(1, onf)

    xs, xt = pl.pallas_call(
        _xsxt_body,
        grid=(n // GS,),
        in_specs=[pl.BlockSpec((GS, inf), lambda i: (i, 0)),
                  pl.BlockSpec((inf, hnf), lambda i: (0, 0)),
                  pl.BlockSpec((inf, hnf), lambda i: (0, 0))],
        out_specs=[pl.BlockSpec((GS, hnf), lambda i: (i, 0)),
                   pl.BlockSpec((GS, hnf), lambda i: (i, 0))],
        out_shape=[jax.ShapeDtypeStruct((n, hnf), jnp.float32),
                   jax.ShapeDtypeStruct((n, hnf), jnp.float32)],
    )(x, ew1[:inf], ew1[inf:2 * inf])

    e_pad = -(-e // (NW * CH)) * (NW * CH)
    row_pad = jnp.pad(row, (0, e_pad - e))
    col_pad = jnp.pad(col, (0, e_pad - e))
    s = _make_sc_gather(e_pad, hnf)(xs, xt, row_pad, col_pad)

    be = 2000
    bcast = lambda i: (0, 0)
    edge_feat = pl.pallas_call(
        _edge_body,
        grid=(e // be,),
        in_specs=[pl.BlockSpec((be, hnf), lambda i: (i, 0)),
                  pl.BlockSpec((be, ein), lambda i: (i, 0)),
                  pl.BlockSpec((ein, hnf), bcast),
                  pl.BlockSpec((1, hnf), bcast),
                  pl.BlockSpec((hnf, hnf), bcast),
                  pl.BlockSpec((1, hnf), bcast),
                  pl.BlockSpec((1, hnf), bcast),
                  pl.BlockSpec((1, hnf), bcast)],
        out_specs=pl.BlockSpec((be, hnf), lambda i: (i, 0)),
        out_shape=jax.ShapeDtypeStruct((e, hnf), jnp.float32),
    )(s, edge_attr, we, eb1r, ew2, eb2r, ngr, nbr)

    agg = jax.ops.segment_sum(edge_feat, row, num_segments=n)

    out = pl.pallas_call(
        _node_body,
        grid=(n // GS,),
        in_specs=[pl.BlockSpec((GS, inf), lambda i: (i, 0)),
                  pl.BlockSpec((GS, hnf), lambda i: (i, 0)),
                  pl.BlockSpec((inf, hnf), bcast),
                  pl.BlockSpec((hnf, hnf), bcast),
                  pl.BlockSpec((hnf, hnf), bcast),
                  pl.BlockSpec((1, hnf), bcast),
                  pl.BlockSpec((hnf, onf), bcast),
                  pl.BlockSpec((1, onf), bcast)],
        out_specs=pl.BlockSpec((GS, onf), lambda i: (i, 0)),
        out_shape=jax.ShapeDtypeStruct((n, onf), jnp.float32),
    )(x, agg, wx, wa, wl, nb1r, nw2, nb2r)

    return (out, edge_feat)
